# baseline (speedup 1.0000x reference)
import jax
import jax.numpy as jnp
from jax.experimental import pallas as pl
from jax.experimental.pallas import tpu as pltpu

_B, _S, _D, _E, _K = 4, 2048, 4096, 64, 8
_N = _B * _S
_R = 1024
_DC = 1024


def _router_kernel(x_ref, w_ref, g_ref, b_ref, t_ref,
                   probs_ref, disp_ref, loss_ref,
                   lacc_ref, acc_load_ref, acc_sq_ref):
    i = pl.program_id(0)
    j = pl.program_id(1)
    ni = pl.num_programs(0)
    nj = pl.num_programs(1)

    part = jax.lax.dot_general(
        w_ref[...], x_ref[...], (((1,), (1,)), ((), ())),
        preferred_element_type=jnp.float32)

    @pl.when(j == 0)
    def _set():
        lacc_ref[...] = part

    @pl.when(j > 0)
    def _add():
        lacc_ref[...] += part

    @pl.when(j == nj - 1)
    def _epilogue():
        logits = lacc_ref[...]
        mu = jnp.mean(logits, axis=0, keepdims=True)
        cen = logits - mu
        var = jnp.mean(cen * cen, axis=0, keepdims=True)
        logits = cen * jax.lax.rsqrt(var + 1e-5) * g_ref[...] + b_ref[...]
        logits = logits / (jnp.abs(t_ref[0, 0]) + 1e-6)

        m = jnp.max(logits, axis=0, keepdims=True)
        ex = jnp.exp(logits - m)
        probs = ex / jnp.sum(ex, axis=0, keepdims=True)
        probs_ref[...] = probs

        eidx = jax.lax.broadcasted_iota(jnp.int32, probs.shape, 0)
        work = probs
        mask = jnp.zeros(probs.shape, jnp.bool_)
        for _ in range(_K):
            cur = jnp.max(work, axis=0, keepdims=True)
            cand = jnp.where(work == cur, eidx, _E)
            sel = jnp.min(cand, axis=0, keepdims=True)
            pick = eidx == sel
            mask = jnp.logical_or(mask, pick)
            work = jnp.where(pick, -jnp.inf, work)

        sel_w = jnp.where(mask, probs, 0.0)
        disp_ref[...] = sel_w / (jnp.sum(sel_w, axis=0, keepdims=True) + 1e-6)

        @pl.when(i == 0)
        def _init():
            acc_load_ref[...] = jnp.zeros_like(acc_load_ref)
            acc_sq_ref[...] = jnp.zeros_like(acc_sq_ref)

        acc_load_ref[...] += jnp.sum(probs, axis=1, keepdims=True)
        acc_sq_ref[...] += jnp.sum(logits * logits, axis=(0, 1), keepdims=True)

        @pl.when(i == ni - 1)
        def _fin():
            n = jnp.float32(_N)
            e = jnp.float32(_E)
            actual = acc_load_ref[...] / n
            ideal = 1.0 / e
            lb = jnp.sum(ideal * (jnp.log(ideal) - jnp.log(actual)),
                         axis=(0, 1), keepdims=True) / e
            z = acc_sq_ref[...] / (n * e)
            loss_ref[...] = 0.01 * z + 0.01 * lb


def kernel(x, W, ln_gamma, ln_beta, temperature):
    b, s, d = x.shape
    flat = x.reshape(-1, d)
    g = ln_gamma.reshape(_E, 1)
    be = ln_beta.reshape(_E, 1)
    t = temperature.reshape(1, 1)
    probs_t, disp_t, loss = pl.pallas_call(
        _router_kernel,
        grid=(_N // _R, _D // _DC),
        in_specs=[
            pl.BlockSpec((_R, _DC), lambda i, j: (i, j)),
            pl.BlockSpec((_E, _DC), lambda i, j: (0, j)),
            pl.BlockSpec((_E, 1), lambda i, j: (0, 0)),
            pl.BlockSpec((_E, 1), lambda i, j: (0, 0)),
            pl.BlockSpec((1, 1), lambda i, j: (0, 0)),
        ],
        out_specs=[
            pl.BlockSpec((_E, _R), lambda i, j: (0, i)),
            pl.BlockSpec((_E, _R), lambda i, j: (0, i)),
            pl.BlockSpec((1, 1), lambda i, j: (0, 0)),
        ],
        out_shape=[
            jax.ShapeDtypeStruct((_E, _N), jnp.float32),
            jax.ShapeDtypeStruct((_E, _N), jnp.float32),
            jax.ShapeDtypeStruct((1, 1), jnp.float32),
        ],
        scratch_shapes=[
            pltpu.VMEM((_E, _R), jnp.float32),
            pltpu.VMEM((_E, 1), jnp.float32),
            pltpu.VMEM((1, 1), jnp.float32),
        ],
    )(flat, W, g, be, t)
    return (probs_t.T, disp_t.T.reshape(b, s, _E), loss[0, 0])


# speedup vs baseline: 1.3431x; 1.3431x over previous
import jax
import jax.numpy as jnp
from jax.experimental import pallas as pl
from jax.experimental.pallas import tpu as pltpu

_B, _S, _D, _E, _K = 4, 2048, 4096, 64, 8
_N = _B * _S
_R = 1024


def _router_kernel(x_ref, w_ref, g_ref, b_ref, t_ref,
                   probs_ref, disp_ref, loss_ref,
                   acc_load_ref, acc_sq_ref):
    step = pl.program_id(0)
    nsteps = pl.num_programs(0)

    xb = x_ref[...]
    w = w_ref[...]
    logits = jax.lax.dot_general(
        w, xb, (((1,), (1,)), ((), ())),
        preferred_element_type=jnp.float32)

    mu = jnp.mean(logits, axis=0, keepdims=True)
    cen = logits - mu
    var = jnp.mean(cen * cen, axis=0, keepdims=True)
    logits = cen * jax.lax.rsqrt(var + 1e-5) * g_ref[...] + b_ref[...]
    logits = logits / (jnp.abs(t_ref[0, 0]) + 1e-6)

    m = jnp.max(logits, axis=0, keepdims=True)
    ex = jnp.exp(logits - m)
    probs = ex / jnp.sum(ex, axis=0, keepdims=True)
    probs_ref[...] = probs

    eidx = jax.lax.broadcasted_iota(jnp.int32, probs.shape, 0)
    work = probs
    mask = jnp.zeros(probs.shape, jnp.bool_)
    for _ in range(_K):
        cur = jnp.max(work, axis=0, keepdims=True)
        cand = jnp.where(work == cur, eidx, _E)
        sel = jnp.min(cand, axis=0, keepdims=True)
        pick = eidx == sel
        mask = jnp.logical_or(mask, pick)
        work = jnp.where(pick, -jnp.inf, work)

    sel_w = jnp.where(mask, probs, 0.0)
    disp_ref[...] = sel_w / (jnp.sum(sel_w, axis=0, keepdims=True) + 1e-6)

    @pl.when(step == 0)
    def _init():
        acc_load_ref[...] = jnp.zeros_like(acc_load_ref)
        acc_sq_ref[...] = jnp.zeros_like(acc_sq_ref)

    acc_load_ref[...] += jnp.sum(probs, axis=1, keepdims=True)
    acc_sq_ref[...] += jnp.sum(logits * logits, axis=(0, 1), keepdims=True)

    @pl.when(step == nsteps - 1)
    def _fin():
        n = jnp.float32(_N)
        e = jnp.float32(_E)
        actual = acc_load_ref[...] / n
        ideal = 1.0 / e
        lb = jnp.sum(ideal * (jnp.log(ideal) - jnp.log(actual)),
                     axis=(0, 1), keepdims=True) / e
        z = acc_sq_ref[...] / (n * e)
        loss_ref[...] = 0.01 * z + 0.01 * lb


def kernel(x, W, ln_gamma, ln_beta, temperature):
    b, s, d = x.shape
    flat = x.reshape(-1, d)
    g = ln_gamma.reshape(_E, 1)
    be = ln_beta.reshape(_E, 1)
    t = temperature.reshape(1, 1)
    probs_t, disp_t, loss = pl.pallas_call(
        _router_kernel,
        grid=(_N // _R,),
        in_specs=[
            pl.BlockSpec((_R, _D), lambda i: (i, 0)),
            pl.BlockSpec((_E, _D), lambda i: (0, 0)),
            pl.BlockSpec((_E, 1), lambda i: (0, 0)),
            pl.BlockSpec((_E, 1), lambda i: (0, 0)),
            pl.BlockSpec((1, 1), lambda i: (0, 0)),
        ],
        out_specs=[
            pl.BlockSpec((_E, _R), lambda i: (0, i)),
            pl.BlockSpec((_E, _R), lambda i: (0, i)),
            pl.BlockSpec((1, 1), lambda i: (0, 0)),
        ],
        out_shape=[
            jax.ShapeDtypeStruct((_E, _N), jnp.float32),
            jax.ShapeDtypeStruct((_E, _N), jnp.float32),
            jax.ShapeDtypeStruct((1, 1), jnp.float32),
        ],
        scratch_shapes=[
            pltpu.VMEM((_E, 1), jnp.float32),
            pltpu.VMEM((1, 1), jnp.float32),
        ],
    )(flat, W, g, be, t)
    return (probs_t.T, disp_t.T.reshape(b, s, _E), loss[0, 0])
